# Initial kernel scaffold; baseline (speedup 1.0000x reference)
import jax
import jax.numpy as jnp
from jax.experimental import pallas as pl


def kernel(x, centroids):
    raise NotImplementedError("write your pallas kernel here")


# speedup vs baseline: 6.4657x; 6.4657x over previous
import functools

import jax
import jax.numpy as jnp
from jax import lax
from jax.experimental import pallas as pl
from jax.experimental.pallas import tpu as pltpu

N, D, K = 8192, 32, 512
BN = 1024


def _assign_body(x_ref, ct_ref, out_ref):
    x = x_ref[...]
    ct = ct_ref[...]
    cn = jnp.sum(ct * ct, axis=0)[None, :]
    dot = lax.dot_general(x, ct, (((1,), (0,)), ((), ())),
                          precision=lax.Precision.HIGHEST,
                          preferred_element_type=jnp.float32)
    s = cn - 2.0 * dot
    m = jnp.min(s, axis=1, keepdims=True)
    iota = lax.broadcasted_iota(jnp.int32, s.shape, 1)
    idx = jnp.min(jnp.where(s == m, iota, K), axis=1)
    out_ref[...] = idx


def kernel(x, centroids):
    assignments = pl.pallas_call(
        _assign_body,
        grid=(N // BN,),
        in_specs=[
            pl.BlockSpec((BN, D), lambda i: (i, 0)),
            pl.BlockSpec((D, K), lambda i: (0, 0)),
        ],
        out_specs=pl.BlockSpec((BN,), lambda i: (i,)),
        out_shape=jax.ShapeDtypeStruct((N,), jnp.int32),
    )(x, centroids.T)
    return (centroids[None, :, :], assignments)
